# baseline (speedup 1.0000x reference)
import functools

import jax
import jax.numpy as jnp
from jax import lax
from jax.experimental import pallas as pl
from jax.experimental.pallas import tpu as pltpu
from jax.experimental.pallas import tpu_sc as plsc

K = 20
B, N, D = 4, 1024, 3
BN = B * N
RB = 1024
NEG = -3.0e38
NW = 32
NPW = BN // NW


def _k1_body(x_ref, xt_ref, w_ref, wb_ref, wd_ref, b1_ref,
             idx_ref, p_ref, q_ref):
    b = pl.program_id(0)
    rb = pl.program_id(1)
    xr = x_ref[0]
    xt = xt_ref[0]
    d2 = jnp.zeros((RB, N), jnp.float32)
    for d in range(D):
        diff = xr[:, d:d + 1] - xt[d:d + 1, :]
        d2 = d2 + diff * diff
    rows = jax.lax.broadcasted_iota(jnp.int32, (RB, N), 0) + rb * RB
    cols = jax.lax.broadcasted_iota(jnp.int32, (RB, N), 1)
    d2 = jnp.where(rows == cols, d2 + 1e10, d2)
    neg = -d2
    r_cols = jnp.sqrt(jnp.sum(xt * xt, axis=0, keepdims=True))
    r_i = jnp.sqrt(jnp.sum(xr * xr, axis=1, keepdims=True))
    ri2 = r_i * r_i
    js, ds, rs = [], [], []
    for _ in range(K):
        m = jnp.max(neg, axis=1, keepdims=True)
        eq = neg == m
        j = jnp.min(jnp.where(eq, cols, jnp.int32(2 ** 30)), axis=1,
                    keepdims=True)
        sel = cols == j
        rj = jnp.max(jnp.where(sel, r_cols, NEG), axis=1, keepdims=True)
        neg = jnp.where(sel, NEG, neg)
        js.append(j)
        ds.append(-m)
        rs.append(rj)
    d2k = jnp.concatenate(ds, axis=1)
    rjm = jnp.concatenate(rs, axis=1)
    dist = jnp.sqrt(d2k)
    dot = 0.5 * (ri2 + rjm * rjm - d2k)
    cos = dot / (r_i * rjm + 1e-8)
    cos = jnp.clip(cos, -1.0 + 1e-7, 1.0 - 1e-7)
    theta = 2.0 * jnp.arctan2(jnp.sqrt((1.0 - cos) * (1.0 + cos)),
                              1.0 + cos)
    f = jnp.concatenate([r_i, rjm, theta, dist], axis=1)
    g = jnp.dot(f, w_ref[...], preferred_element_type=jnp.float32)
    h = g[:, :64]
    for k in range(1, K):
        h = jnp.maximum(h, g[:, 64 * k:64 * (k + 1)])
    idx_ref[0] = jnp.concatenate(js, axis=1) + b * N
    p_ref[0] = jnp.dot(h, wb_ref[...], preferred_element_type=jnp.float32)
    q_ref[0] = jnp.dot(h, wd_ref[...],
                       preferred_element_type=jnp.float32) + b1_ref[...]


def _k1(x, xt, w_rri, w1b, w1d, b1):
    return pl.pallas_call(
        _k1_body,
        grid=(B, N // RB),
        in_specs=[
            pl.BlockSpec((1, RB, D), lambda b, r: (b, r, 0)),
            pl.BlockSpec((1, D, N), lambda b, r: (b, 0, 0)),
            pl.BlockSpec((3 * K + 1, 64 * K), lambda b, r: (0, 0)),
            pl.BlockSpec((64, 128), lambda b, r: (0, 0)),
            pl.BlockSpec((64, 128), lambda b, r: (0, 0)),
            pl.BlockSpec((1, 128), lambda b, r: (0, 0)),
        ],
        out_specs=[
            pl.BlockSpec((1, RB, K), lambda b, r: (b, r, 0)),
            pl.BlockSpec((1, RB, 128), lambda b, r: (b, r, 0)),
            pl.BlockSpec((1, RB, 128), lambda b, r: (b, r, 0)),
        ],
        out_shape=[
            jax.ShapeDtypeStruct((B, N, K), jnp.int32),
            jax.ShapeDtypeStruct((B, N, 128), jnp.float32),
            jax.ShapeDtypeStruct((B, N, 128), jnp.float32),
        ],
    )(x, xt, w_rri, w1b, w1d, b1)


def _k3_sc_body(p_hbm, idxt_hbm, out_hbm, idx_v, rows_a, rows_b, acc_v,
                sem_a, sem_b):
    wid = lax.axis_index("s") * 2 + lax.axis_index("c")
    base = wid * NPW
    pltpu.sync_copy(idxt_hbm.at[wid], idx_v)
    bufs = (rows_a, rows_b)
    sems = (sem_a, sem_b)
    pltpu.async_copy(p_hbm.at[idx_v.at[0]], acc_v, sem_a).wait()
    cps = {1: pltpu.async_copy(p_hbm.at[idx_v.at[1]], bufs[1 % 2],
                               sems[1 % 2])}
    for k in range(1, K):
        if k + 1 < K:
            cps[k + 1] = pltpu.async_copy(p_hbm.at[idx_v.at[k + 1]],
                                          bufs[(k + 1) % 2], sems[(k + 1) % 2])
        cps[k].wait()
        rows_v = bufs[k % 2]

        def body(r, _):
            for c in range(8):
                sl = pl.ds(c * 16, 16)
                acc_v[r, sl] = jnp.maximum(acc_v[r, sl], rows_v[r, sl])
            return 0

        lax.fori_loop(0, NPW, body, 0)
    pltpu.sync_copy(acc_v, out_hbm.at[pl.ds(base, NPW)])


def _k3_sc(p, idxt):
    mesh = plsc.VectorSubcoreMesh(core_axis_name="c", subcore_axis_name="s")
    f = functools.partial(
        pl.kernel,
        out_type=jax.ShapeDtypeStruct((BN, 128), jnp.float32),
        mesh=mesh,
        scratch_types=[
            pltpu.VMEM((K, NPW), jnp.int32),
            pltpu.VMEM((NPW, 128), jnp.float32),
            pltpu.VMEM((NPW, 128), jnp.float32),
            pltpu.VMEM((NPW, 128), jnp.float32),
            pltpu.SemaphoreType.DMA,
            pltpu.SemaphoreType.DMA,
        ],
    )(_k3_sc_body)
    return f(p, idxt)


def _k5_sc_body(ss_hbm, idxt_hbm, num_hbm, den_hbm,
                idx_v, rows_a, rows_b, own_v, res_v, sem_a, sem_b):
    wid = lax.axis_index("s") * 2 + lax.axis_index("c")
    base = wid * NPW
    pltpu.sync_copy(idxt_hbm.at[wid], idx_v)
    pltpu.sync_copy(ss_hbm.at[pl.ds(base, NPW)], own_v)
    bufs = (rows_a, rows_b)
    sems = (sem_a, sem_b)
    an = jnp.zeros((16,), jnp.float32)
    ad = jnp.zeros((16,), jnp.float32)
    cps = {0: pltpu.async_copy(ss_hbm.at[idx_v.at[0]], bufs[0], sems[0])}
    for k in range(K):
        if k + 1 < K:
            cps[k + 1] = pltpu.async_copy(ss_hbm.at[idx_v.at[k + 1]],
                                          bufs[(k + 1) % 2], sems[(k + 1) % 2])
        cps[k].wait()
        rows_v = bufs[k % 2]

        def body(r, carry):
            an_, ad_ = carry
            for c in range(2):
                sl = pl.ds(c * 16, 16)
                g = rows_v[r, sl]
                o = own_v[r, sl]
                an_ = an_ + g * o
                ad_ = ad_ + g * g
            return an_, ad_

        an, ad = lax.fori_loop(0, NPW, body, (an, ad))
    res_v[0, :] = an
    res_v[1, :] = ad
    pltpu.sync_copy(res_v.at[0], num_hbm.at[wid])
    pltpu.sync_copy(res_v.at[1], den_hbm.at[wid])


def _k5_sc(ssoft, idxt):
    mesh = plsc.VectorSubcoreMesh(core_axis_name="c", subcore_axis_name="s")
    f = functools.partial(
        pl.kernel,
        out_type=[
            jax.ShapeDtypeStruct((NW, 16), jnp.float32),
            jax.ShapeDtypeStruct((NW, 16), jnp.float32),
        ],
        mesh=mesh,
        scratch_types=[
            pltpu.VMEM((K, NPW), jnp.int32),
            pltpu.VMEM((NPW, 128), jnp.float32),
            pltpu.VMEM((NPW, 128), jnp.float32),
            pltpu.VMEM((NPW, 128), jnp.float32),
            pltpu.VMEM((2, 16), jnp.float32),
            pltpu.SemaphoreType.DMA,
            pltpu.SemaphoreType.DMA,
        ],
    )(_k5_sc_body)
    return f(ssoft, idxt)


def _ortho_stats(ssoft, csize):
    ss = lax.dot_general(ssoft, ssoft, (((0,), (0,)), ((), ())),
                         preferred_element_type=jnp.float32)
    frob = jnp.sqrt(jnp.sum(ss * ss, keepdims=True)) + 1e-15
    ceye = jnp.where(
        jax.lax.broadcasted_iota(jnp.int32, (csize, csize), 0)
        == jax.lax.broadcasted_iota(jnp.int32, (csize, csize), 1),
        1.0 / jnp.sqrt(jnp.float32(csize)), 0.0)
    diff = ss / frob - ceye
    return jnp.sqrt(jnp.sum(diff * diff, keepdims=True))


def _k4_body(mg_ref, q_ref, wp_ref, bp_ref, pool_ref, ssoft_ref, o_ref):
    xd = mg_ref[0] + q_ref[0]
    s = jnp.dot(xd, wp_ref[...], preferred_element_type=jnp.float32) + bp_ref[...]
    s = s - jnp.max(s, axis=1, keepdims=True)
    e = jnp.exp(s)
    ssoft = e / jnp.sum(e, axis=1, keepdims=True)
    pool_ref[0] = lax.dot_general(ssoft, xd, (((0,), (0,)), ((), ())),
                                  preferred_element_type=jnp.float32)
    ssoft_ref[0] = jnp.concatenate(
        [ssoft, jnp.zeros((N, 96), jnp.float32)], axis=1)
    o_ref[0] = _ortho_stats(ssoft, 32)


def _k4(mg, q, wp1, bp1):
    return pl.pallas_call(
        _k4_body,
        grid=(B,),
        in_specs=[
            pl.BlockSpec((1, N, 128), lambda b: (b, 0, 0)),
            pl.BlockSpec((1, N, 128), lambda b: (b, 0, 0)),
            pl.BlockSpec((128, 32), lambda b: (0, 0)),
            pl.BlockSpec((1, 32), lambda b: (0, 0)),
        ],
        out_specs=[
            pl.BlockSpec((1, 32, 128), lambda b: (b, 0, 0)),
            pl.BlockSpec((1, N, 128), lambda b: (b, 0, 0)),
            pl.BlockSpec((1, 1, 1), lambda b: (b, 0, 0)),
        ],
        out_shape=[
            jax.ShapeDtypeStruct((B, 32, 128), jnp.float32),
            jax.ShapeDtypeStruct((B, N, 128), jnp.float32),
            jax.ShapeDtypeStruct((B, 1, 1), jnp.float32),
        ],
    )(mg, q, wp1, bp1)


def _pool_stats(ssoft, m, csize):
    t = jnp.dot(m, ssoft, preferred_element_type=jnp.float32)
    num = jnp.sum(t * ssoft, keepdims=True)
    ssq_row = jnp.sum(ssoft * ssoft, axis=1, keepdims=True)
    deg = jnp.sum(m, axis=0, keepdims=True)
    den = jnp.dot(deg, ssq_row, preferred_element_type=jnp.float32) + 1e-15
    o = _ortho_stats(ssoft, csize)
    return num, den, o


def _k6_body(x_ref, wb_ref, wd_ref, b_ref, wp_ref, bp_ref,
             pool_ref, num_ref, den_ref, o_ref):
    x = x_ref[0]
    n = 32
    diff = x[:, None, :] - x[None, :, :]
    d2 = jnp.sum(diff * diff, axis=-1)
    rows = jax.lax.broadcasted_iota(jnp.int32, (n, n), 0)
    cols = jax.lax.broadcasted_iota(jnp.int32, (n, n), 1)
    d2 = jnp.where(rows == cols, d2 + 1e10, d2)
    neg = -d2
    macc = jnp.zeros((n, n), jnp.float32)
    for _ in range(K):
        m = jnp.max(neg, axis=1, keepdims=True)
        eq = neg == m
        j = jnp.min(jnp.where(eq, cols, jnp.int32(2 ** 30)), axis=1,
                    keepdims=True)
        sel = cols == j
        neg = jnp.where(sel, NEG, neg)
        macc = jnp.where(sel, 1.0, macc)
    p = jnp.dot(x, wb_ref[...], preferred_element_type=jnp.float32)
    q = jnp.dot(x, wd_ref[...], preferred_element_type=jnp.float32) + b_ref[...]
    mx = jnp.max(jnp.where(macc[:, :, None] > 0, p[None, :, :], NEG), axis=1)
    h2 = q + mx
    s = jnp.dot(h2, wp_ref[...], preferred_element_type=jnp.float32) + bp_ref[...]
    s = s - jnp.max(s, axis=1, keepdims=True)
    e = jnp.exp(s)
    ssoft = e / jnp.sum(e, axis=1, keepdims=True)
    pool_ref[0] = lax.dot_general(ssoft, h2, (((0,), (0,)), ((), ())),
                                  preferred_element_type=jnp.float32)
    num, den, o = _pool_stats(ssoft, macc, 8)
    num_ref[0] = num
    den_ref[0] = den
    o_ref[0] = o


def _k6(xd, w2b, w2d, b2, wp2, bp2):
    return pl.pallas_call(
        _k6_body,
        grid=(B,),
        in_specs=[
            pl.BlockSpec((1, 32, 128), lambda b: (b, 0, 0)),
            pl.BlockSpec((128, 256), lambda b: (0, 0)),
            pl.BlockSpec((128, 256), lambda b: (0, 0)),
            pl.BlockSpec((1, 256), lambda b: (0, 0)),
            pl.BlockSpec((256, 8), lambda b: (0, 0)),
            pl.BlockSpec((1, 8), lambda b: (0, 0)),
        ],
        out_specs=[
            pl.BlockSpec((1, 8, 256), lambda b: (b, 0, 0)),
            pl.BlockSpec((1, 1, 1), lambda b: (b, 0, 0)),
            pl.BlockSpec((1, 1, 1), lambda b: (b, 0, 0)),
            pl.BlockSpec((1, 1, 1), lambda b: (b, 0, 0)),
        ],
        out_shape=[
            jax.ShapeDtypeStruct((B, 8, 256), jnp.float32),
            jax.ShapeDtypeStruct((B, 1, 1), jnp.float32),
            jax.ShapeDtypeStruct((B, 1, 1), jnp.float32),
            jax.ShapeDtypeStruct((B, 1, 1), jnp.float32),
        ],
    )(xd, w2b, w2d, b2, wp2, bp2)


def _k7_body(x_ref, wb_ref, wd_ref, b_ref, wf1_ref, bf1_ref, wf2_ref,
             bf2_ref, wf3_ref, bf3_ref, out_ref):
    x = x_ref[...].reshape(B * 8, 256)
    p = jnp.dot(x, wb_ref[...], preferred_element_type=jnp.float32)
    q = jnp.dot(x, wd_ref[...], preferred_element_type=jnp.float32) + b_ref[...]
    pooled_rows = []
    rows3 = jax.lax.broadcasted_iota(jnp.int32, (8, 8, 1024), 0)
    cols3 = jax.lax.broadcasted_iota(jnp.int32, (8, 8, 1024), 1)
    offdiag = rows3 != cols3
    for b in range(B):
        pb = p[b * 8:(b + 1) * 8, :]
        mx = jnp.max(jnp.where(offdiag, pb[None, :, :], NEG),
                     axis=1)
        h3 = q[b * 8:(b + 1) * 8, :] + mx
        pooled_rows.append(jnp.sum(h3, axis=0, keepdims=True))
    pooled = jnp.concatenate(pooled_rows, axis=0)
    h4 = jnp.maximum(
        jnp.dot(pooled, wf1_ref[...], preferred_element_type=jnp.float32)
        + bf1_ref[...], 0.0)
    h5 = jnp.maximum(
        jnp.dot(h4, wf2_ref[...], preferred_element_type=jnp.float32)
        + bf2_ref[...], 0.0)
    h6 = jnp.dot(h5, wf3_ref[...], preferred_element_type=jnp.float32) \
        + bf3_ref[...]
    h6 = h6 - jnp.max(h6, axis=1, keepdims=True)
    e = jnp.exp(h6)
    out_ref[...] = e / jnp.sum(e, axis=1, keepdims=True)


def _k7(xd, w3b, w3d, b3, wf1, bf1, wf2, bf2, wf3, bf3):
    full = lambda *shape: pl.BlockSpec(shape, lambda: tuple(0 for _ in shape))
    return pl.pallas_call(
        _k7_body,
        grid=(),
        in_specs=[
            full(B, 8, 256),
            full(256, 1024), full(256, 1024), full(1, 1024),
            full(1024, 512), full(1, 512),
            full(512, 256), full(1, 256),
            full(256, 40), full(1, 40),
        ],
        out_specs=full(B, 40),
        out_shape=jax.ShapeDtypeStruct((B, 40), jnp.float32),
    )(xd, w3b, w3d, b3, wf1, bf1, wf2, bf2, wf3, bf3)


def kernel(x, W_rri, W1, b1, Wp1, bp1, W2, b2, Wp2, bp2, W3, b3, Wp3, bp3,
           Wf1, bf1, Wf2, bf2, Wf3, bf3):
    xt = jnp.transpose(x, (0, 2, 1))
    w1b, w1d = W1[64:], W1[:64] - W1[64:]
    eyek = jnp.eye(K, dtype=jnp.float32)
    w_big = jnp.concatenate(
        [jnp.tile(W_rri[0:1, :], (1, K)),
         jnp.kron(eyek, W_rri[1:2, :]),
         jnp.kron(eyek, W_rri[2:3, :]),
         jnp.kron(eyek, W_rri[3:4, :])], axis=0)
    idx, p1, q1 = _k1(x, xt, w_big, w1b, w1d, b1.reshape(1, -1))
    idxt = jnp.transpose(idx.reshape(NW, NPW, K), (0, 2, 1))
    mg = _k3_sc(p1.reshape(BN, 128), idxt)
    pool1, ssoft1, o1 = _k4(mg.reshape(B, N, 128), q1, Wp1, bp1.reshape(1, -1))
    num1, den1 = _k5_sc(ssoft1.reshape(BN, 128), idxt)
    w2b, w2d = W2[128:], W2[:128] - W2[128:]
    pool2, num2, den2, o2 = _k6(pool1, w2b, w2d, b2.reshape(1, -1), Wp2,
                                bp2.reshape(1, -1))
    w3b, w3d = W3[256:], W3[:256] - W3[256:]
    out = _k7(pool2, w3b, w3d, b3.reshape(1, -1), Wf1, bf1.reshape(1, -1),
              Wf2, bf2.reshape(1, -1), Wf3, bf3.reshape(1, -1))
    num1_b = jnp.sum(num1.reshape(B, 8 * 16), axis=1)
    den1_b = jnp.sum(den1.reshape(B, 8 * 16), axis=1) + 1e-15
    mc = (jnp.mean(-(num1_b / den1_b))
          + jnp.mean(-(num2[:, 0, 0] / den2[:, 0, 0])) - 1.0)
    o = jnp.mean(o1[:, 0, 0]) + jnp.mean(o2[:, 0, 0]) + 0.0
    return out, mc, o


# speedup vs baseline: 1.2139x; 1.2139x over previous
import functools

import jax
import jax.numpy as jnp
from jax import lax
from jax.experimental import pallas as pl
from jax.experimental.pallas import tpu as pltpu
from jax.experimental.pallas import tpu_sc as plsc

K = 20
B, N, D = 4, 1024, 3
BN = B * N
RB = 512
NEG = -3.0e38
NW = 32
NPW = BN // NW


def _k1_body(x_ref, xt_ref, w_ref, wb_ref, wd_ref, b1_ref,
             idx_ref, p_ref, q_ref):
    b = pl.program_id(0)
    rb = pl.program_id(1)
    xr = x_ref[0]
    xt = xt_ref[0]
    d2 = jnp.zeros((RB, N), jnp.float32)
    for d in range(D):
        diff = xr[:, d:d + 1] - xt[d:d + 1, :]
        d2 = d2 + diff * diff
    rows = jax.lax.broadcasted_iota(jnp.int32, (RB, N), 0) + rb * RB
    cols = jax.lax.broadcasted_iota(jnp.int32, (RB, N), 1)
    d2 = jnp.where(rows == cols, d2 + 1e10, d2)
    neg = -d2
    r_cols = jnp.sqrt(jnp.sum(xt * xt, axis=0, keepdims=True))
    r_i = jnp.sqrt(jnp.sum(xr * xr, axis=1, keepdims=True))
    ri2 = r_i * r_i
    js, ds, rs = [], [], []
    for _ in range(K):
        m = jnp.max(neg, axis=1, keepdims=True)
        eq = neg == m
        j = jnp.min(jnp.where(eq, cols, jnp.int32(2 ** 30)), axis=1,
                    keepdims=True)
        sel = cols == j
        rj = jnp.max(jnp.where(sel, r_cols, NEG), axis=1, keepdims=True)
        neg = jnp.where(sel, NEG, neg)
        js.append(j)
        ds.append(-m)
        rs.append(rj)
    d2k = jnp.concatenate(ds, axis=1)
    rjm = jnp.concatenate(rs, axis=1)
    dist = jnp.sqrt(d2k)
    dot = 0.5 * (ri2 + rjm * rjm - d2k)
    cos = dot / (r_i * rjm + 1e-8)
    cos = jnp.clip(cos, -1.0 + 1e-7, 1.0 - 1e-7)
    theta = 2.0 * jnp.arctan2(jnp.sqrt((1.0 - cos) * (1.0 + cos)),
                              1.0 + cos)
    f = jnp.concatenate([r_i, rjm, theta, dist], axis=1)
    g = jnp.dot(f, w_ref[...], preferred_element_type=jnp.float32)
    h = g[:, :64]
    for k in range(1, K):
        h = jnp.maximum(h, g[:, 64 * k:64 * (k + 1)])
    idx_ref[0] = jnp.concatenate(js, axis=1) + b * N
    p_ref[0] = jnp.dot(h, wb_ref[...], preferred_element_type=jnp.float32)
    q_ref[0] = jnp.dot(h, wd_ref[...],
                       preferred_element_type=jnp.float32) + b1_ref[...]


def _k1(x, xt, w_rri, w1b, w1d, b1):
    return pl.pallas_call(
        _k1_body,
        grid=(B, N // RB),
        in_specs=[
            pl.BlockSpec((1, RB, D), lambda b, r: (b, r, 0)),
            pl.BlockSpec((1, D, N), lambda b, r: (b, 0, 0)),
            pl.BlockSpec((3 * K + 1, 64 * K), lambda b, r: (0, 0)),
            pl.BlockSpec((64, 128), lambda b, r: (0, 0)),
            pl.BlockSpec((64, 128), lambda b, r: (0, 0)),
            pl.BlockSpec((1, 128), lambda b, r: (0, 0)),
        ],
        out_specs=[
            pl.BlockSpec((1, RB, K), lambda b, r: (b, r, 0)),
            pl.BlockSpec((1, RB, 128), lambda b, r: (b, r, 0)),
            pl.BlockSpec((1, RB, 128), lambda b, r: (b, r, 0)),
        ],
        out_shape=[
            jax.ShapeDtypeStruct((B, N, K), jnp.int32),
            jax.ShapeDtypeStruct((B, N, 128), jnp.float32),
            jax.ShapeDtypeStruct((B, N, 128), jnp.float32),
        ],
    )(x, xt, w_rri, w1b, w1d, b1)


def _k3_sc_body(p_hbm, idxt_hbm, out_hbm, idx_v, rows_a, rows_b, acc_v,
                sem_a, sem_b):
    wid = lax.axis_index("s") * 2 + lax.axis_index("c")
    base = wid * NPW
    pltpu.sync_copy(idxt_hbm.at[wid], idx_v)
    bufs = (rows_a, rows_b)
    sems = (sem_a, sem_b)
    pltpu.async_copy(p_hbm.at[idx_v.at[0]], acc_v, sem_a).wait()
    cps = {1: pltpu.async_copy(p_hbm.at[idx_v.at[1]], bufs[1 % 2],
                               sems[1 % 2])}
    for k in range(1, K):
        if k + 1 < K:
            cps[k + 1] = pltpu.async_copy(p_hbm.at[idx_v.at[k + 1]],
                                          bufs[(k + 1) % 2], sems[(k + 1) % 2])
        cps[k].wait()
        rows_v = bufs[k % 2]

        def body(r, _):
            for c in range(8):
                sl = pl.ds(c * 16, 16)
                acc_v[r, sl] = jnp.maximum(acc_v[r, sl], rows_v[r, sl])
            return 0

        lax.fori_loop(0, NPW, body, 0)
    pltpu.sync_copy(acc_v, out_hbm.at[pl.ds(base, NPW)])


def _k3_sc(p, idxt):
    mesh = plsc.VectorSubcoreMesh(core_axis_name="c", subcore_axis_name="s")
    f = functools.partial(
        pl.kernel,
        out_type=jax.ShapeDtypeStruct((BN, 128), jnp.float32),
        mesh=mesh,
        scratch_types=[
            pltpu.VMEM((K, NPW), jnp.int32),
            pltpu.VMEM((NPW, 128), jnp.float32),
            pltpu.VMEM((NPW, 128), jnp.float32),
            pltpu.VMEM((NPW, 128), jnp.float32),
            pltpu.SemaphoreType.DMA,
            pltpu.SemaphoreType.DMA,
        ],
    )(_k3_sc_body)
    return f(p, idxt)


def _k5_sc_body(ss_hbm, idxt_hbm, num_hbm, den_hbm,
                idx_v, rows_a, rows_b, own_v, res_v, sem_a, sem_b):
    wid = lax.axis_index("s") * 2 + lax.axis_index("c")
    base = wid * NPW
    pltpu.sync_copy(idxt_hbm.at[wid], idx_v)
    pltpu.sync_copy(ss_hbm.at[pl.ds(base, NPW)], own_v)
    bufs = (rows_a, rows_b)
    sems = (sem_a, sem_b)
    an = jnp.zeros((16,), jnp.float32)
    ad = jnp.zeros((16,), jnp.float32)
    cps = {0: pltpu.async_copy(ss_hbm.at[idx_v.at[0]], bufs[0], sems[0])}
    for k in range(K):
        if k + 1 < K:
            cps[k + 1] = pltpu.async_copy(ss_hbm.at[idx_v.at[k + 1]],
                                          bufs[(k + 1) % 2], sems[(k + 1) % 2])
        cps[k].wait()
        rows_v = bufs[k % 2]

        def body(r, carry):
            an_, ad_ = carry
            for c in range(2):
                sl = pl.ds(c * 16, 16)
                g = rows_v[r, sl]
                o = own_v[r, sl]
                an_ = an_ + g * o
                ad_ = ad_ + g * g
            return an_, ad_

        an, ad = lax.fori_loop(0, NPW, body, (an, ad))
    res_v[0, :] = an
    res_v[1, :] = ad
    pltpu.sync_copy(res_v.at[0], num_hbm.at[wid])
    pltpu.sync_copy(res_v.at[1], den_hbm.at[wid])


def _k5_sc(ssoft, idxt):
    mesh = plsc.VectorSubcoreMesh(core_axis_name="c", subcore_axis_name="s")
    f = functools.partial(
        pl.kernel,
        out_type=[
            jax.ShapeDtypeStruct((NW, 16), jnp.float32),
            jax.ShapeDtypeStruct((NW, 16), jnp.float32),
        ],
        mesh=mesh,
        scratch_types=[
            pltpu.VMEM((K, NPW), jnp.int32),
            pltpu.VMEM((NPW, 128), jnp.float32),
            pltpu.VMEM((NPW, 128), jnp.float32),
            pltpu.VMEM((NPW, 128), jnp.float32),
            pltpu.VMEM((2, 16), jnp.float32),
            pltpu.SemaphoreType.DMA,
            pltpu.SemaphoreType.DMA,
        ],
    )(_k5_sc_body)
    return f(ssoft, idxt)


def _ortho_stats(ssoft, csize):
    ss = lax.dot_general(ssoft, ssoft, (((0,), (0,)), ((), ())),
                         preferred_element_type=jnp.float32)
    frob = jnp.sqrt(jnp.sum(ss * ss, keepdims=True)) + 1e-15
    ceye = jnp.where(
        jax.lax.broadcasted_iota(jnp.int32, (csize, csize), 0)
        == jax.lax.broadcasted_iota(jnp.int32, (csize, csize), 1),
        1.0 / jnp.sqrt(jnp.float32(csize)), 0.0)
    diff = ss / frob - ceye
    return jnp.sqrt(jnp.sum(diff * diff, keepdims=True))


def _k4_body(mg_ref, q_ref, wp_ref, bp_ref, pool_ref, ssoft_ref, o_ref):
    xd = mg_ref[0] + q_ref[0]
    s = jnp.dot(xd, wp_ref[...], preferred_element_type=jnp.float32) + bp_ref[...]
    s = s - jnp.max(s, axis=1, keepdims=True)
    e = jnp.exp(s)
    ssoft = e / jnp.sum(e, axis=1, keepdims=True)
    pool_ref[0] = lax.dot_general(ssoft, xd, (((0,), (0,)), ((), ())),
                                  preferred_element_type=jnp.float32)
    ssoft_ref[0] = jnp.concatenate(
        [ssoft, jnp.zeros((N, 96), jnp.float32)], axis=1)
    o_ref[0] = _ortho_stats(ssoft, 32)


def _k4(mg, q, wp1, bp1):
    return pl.pallas_call(
        _k4_body,
        grid=(B,),
        in_specs=[
            pl.BlockSpec((1, N, 128), lambda b: (b, 0, 0)),
            pl.BlockSpec((1, N, 128), lambda b: (b, 0, 0)),
            pl.BlockSpec((128, 32), lambda b: (0, 0)),
            pl.BlockSpec((1, 32), lambda b: (0, 0)),
        ],
        out_specs=[
            pl.BlockSpec((1, 32, 128), lambda b: (b, 0, 0)),
            pl.BlockSpec((1, N, 128), lambda b: (b, 0, 0)),
            pl.BlockSpec((1, 1, 1), lambda b: (b, 0, 0)),
        ],
        out_shape=[
            jax.ShapeDtypeStruct((B, 32, 128), jnp.float32),
            jax.ShapeDtypeStruct((B, N, 128), jnp.float32),
            jax.ShapeDtypeStruct((B, 1, 1), jnp.float32),
        ],
    )(mg, q, wp1, bp1)


def _pool_stats(ssoft, m, csize):
    t = jnp.dot(m, ssoft, preferred_element_type=jnp.float32)
    num = jnp.sum(t * ssoft, keepdims=True)
    ssq_row = jnp.sum(ssoft * ssoft, axis=1, keepdims=True)
    deg = jnp.sum(m, axis=0, keepdims=True)
    den = jnp.dot(deg, ssq_row, preferred_element_type=jnp.float32) + 1e-15
    o = _ortho_stats(ssoft, csize)
    return num, den, o


def _k6_body(x_ref, wb_ref, wd_ref, b_ref, wp_ref, bp_ref,
             pool_ref, num_ref, den_ref, o_ref):
    x = x_ref[0]
    n = 32
    diff = x[:, None, :] - x[None, :, :]
    d2 = jnp.sum(diff * diff, axis=-1)
    rows = jax.lax.broadcasted_iota(jnp.int32, (n, n), 0)
    cols = jax.lax.broadcasted_iota(jnp.int32, (n, n), 1)
    d2 = jnp.where(rows == cols, d2 + 1e10, d2)
    neg = -d2
    macc = jnp.zeros((n, n), jnp.float32)
    for _ in range(K):
        m = jnp.max(neg, axis=1, keepdims=True)
        eq = neg == m
        j = jnp.min(jnp.where(eq, cols, jnp.int32(2 ** 30)), axis=1,
                    keepdims=True)
        sel = cols == j
        neg = jnp.where(sel, NEG, neg)
        macc = jnp.where(sel, 1.0, macc)
    p = jnp.dot(x, wb_ref[...], preferred_element_type=jnp.float32)
    q = jnp.dot(x, wd_ref[...], preferred_element_type=jnp.float32) + b_ref[...]
    mx = jnp.max(jnp.where(macc[:, :, None] > 0, p[None, :, :], NEG), axis=1)
    h2 = q + mx
    s = jnp.dot(h2, wp_ref[...], preferred_element_type=jnp.float32) + bp_ref[...]
    s = s - jnp.max(s, axis=1, keepdims=True)
    e = jnp.exp(s)
    ssoft = e / jnp.sum(e, axis=1, keepdims=True)
    pool_ref[0] = lax.dot_general(ssoft, h2, (((0,), (0,)), ((), ())),
                                  preferred_element_type=jnp.float32)
    num, den, o = _pool_stats(ssoft, macc, 8)
    num_ref[0] = num
    den_ref[0] = den
    o_ref[0] = o


def _k6(xd, w2b, w2d, b2, wp2, bp2):
    return pl.pallas_call(
        _k6_body,
        grid=(B,),
        in_specs=[
            pl.BlockSpec((1, 32, 128), lambda b: (b, 0, 0)),
            pl.BlockSpec((128, 256), lambda b: (0, 0)),
            pl.BlockSpec((128, 256), lambda b: (0, 0)),
            pl.BlockSpec((1, 256), lambda b: (0, 0)),
            pl.BlockSpec((256, 8), lambda b: (0, 0)),
            pl.BlockSpec((1, 8), lambda b: (0, 0)),
        ],
        out_specs=[
            pl.BlockSpec((1, 8, 256), lambda b: (b, 0, 0)),
            pl.BlockSpec((1, 1, 1), lambda b: (b, 0, 0)),
            pl.BlockSpec((1, 1, 1), lambda b: (b, 0, 0)),
            pl.BlockSpec((1, 1, 1), lambda b: (b, 0, 0)),
        ],
        out_shape=[
            jax.ShapeDtypeStruct((B, 8, 256), jnp.float32),
            jax.ShapeDtypeStruct((B, 1, 1), jnp.float32),
            jax.ShapeDtypeStruct((B, 1, 1), jnp.float32),
            jax.ShapeDtypeStruct((B, 1, 1), jnp.float32),
        ],
    )(xd, w2b, w2d, b2, wp2, bp2)


def _k7_body(x_ref, wb_ref, wd_ref, b_ref, wf1_ref, bf1_ref, wf2_ref,
             bf2_ref, wf3_ref, bf3_ref, out_ref):
    x = x_ref[...].reshape(B * 8, 256)
    p = jnp.dot(x, wb_ref[...], preferred_element_type=jnp.float32)
    q = jnp.dot(x, wd_ref[...], preferred_element_type=jnp.float32) + b_ref[...]
    pooled_rows = []
    rows3 = jax.lax.broadcasted_iota(jnp.int32, (8, 8, 1024), 0)
    cols3 = jax.lax.broadcasted_iota(jnp.int32, (8, 8, 1024), 1)
    offdiag = rows3 != cols3
    for b in range(B):
        pb = p[b * 8:(b + 1) * 8, :]
        mx = jnp.max(jnp.where(offdiag, pb[None, :, :], NEG),
                     axis=1)
        h3 = q[b * 8:(b + 1) * 8, :] + mx
        pooled_rows.append(jnp.sum(h3, axis=0, keepdims=True))
    pooled = jnp.concatenate(pooled_rows, axis=0)
    h4 = jnp.maximum(
        jnp.dot(pooled, wf1_ref[...], preferred_element_type=jnp.float32)
        + bf1_ref[...], 0.0)
    h5 = jnp.maximum(
        jnp.dot(h4, wf2_ref[...], preferred_element_type=jnp.float32)
        + bf2_ref[...], 0.0)
    h6 = jnp.dot(h5, wf3_ref[...], preferred_element_type=jnp.float32) \
        + bf3_ref[...]
    h6 = h6 - jnp.max(h6, axis=1, keepdims=True)
    e = jnp.exp(h6)
    out_ref[...] = e / jnp.sum(e, axis=1, keepdims=True)


def _k7(xd, w3b, w3d, b3, wf1, bf1, wf2, bf2, wf3, bf3):
    full = lambda *shape: pl.BlockSpec(shape, lambda: tuple(0 for _ in shape))
    return pl.pallas_call(
        _k7_body,
        grid=(),
        in_specs=[
            full(B, 8, 256),
            full(256, 1024), full(256, 1024), full(1, 1024),
            full(1024, 512), full(1, 512),
            full(512, 256), full(1, 256),
            full(256, 40), full(1, 40),
        ],
        out_specs=full(B, 40),
        out_shape=jax.ShapeDtypeStruct((B, 40), jnp.float32),
    )(xd, w3b, w3d, b3, wf1, bf1, wf2, bf2, wf3, bf3)


def kernel(x, W_rri, W1, b1, Wp1, bp1, W2, b2, Wp2, bp2, W3, b3, Wp3, bp3,
           Wf1, bf1, Wf2, bf2, Wf3, bf3):
    xt = jnp.transpose(x, (0, 2, 1))
    w1b, w1d = W1[64:], W1[:64] - W1[64:]
    eyek = jnp.eye(K, dtype=jnp.float32)
    w_big = jnp.concatenate(
        [jnp.tile(W_rri[0:1, :], (1, K)),
         jnp.kron(eyek, W_rri[1:2, :]),
         jnp.kron(eyek, W_rri[2:3, :]),
         jnp.kron(eyek, W_rri[3:4, :])], axis=0)
    idx, p1, q1 = _k1(x, xt, w_big, w1b, w1d, b1.reshape(1, -1))
    idxt = jnp.transpose(idx.reshape(NW, NPW, K), (0, 2, 1))
    mg = _k3_sc(p1.reshape(BN, 128), idxt)
    pool1, ssoft1, o1 = _k4(mg.reshape(B, N, 128), q1, Wp1, bp1.reshape(1, -1))
    num1, den1 = _k5_sc(ssoft1.reshape(BN, 128), idxt)
    w2b, w2d = W2[128:], W2[:128] - W2[128:]
    pool2, num2, den2, o2 = _k6(pool1, w2b, w2d, b2.reshape(1, -1), Wp2,
                                bp2.reshape(1, -1))
    w3b, w3d = W3[256:], W3[:256] - W3[256:]
    out = _k7(pool2, w3b, w3d, b3.reshape(1, -1), Wf1, bf1.reshape(1, -1),
              Wf2, bf2.reshape(1, -1), Wf3, bf3.reshape(1, -1))
    num1_b = jnp.sum(num1.reshape(B, 8 * 16), axis=1)
    den1_b = jnp.sum(den1.reshape(B, 8 * 16), axis=1) + 1e-15
    mc = (jnp.mean(-(num1_b / den1_b))
          + jnp.mean(-(num2[:, 0, 0] / den2[:, 0, 0])) - 1.0)
    o = jnp.mean(o1[:, 0, 0]) + jnp.mean(o2[:, 0, 0]) + 0.0
    return out, mc, o


# speedup vs baseline: 1.2205x; 1.0054x over previous
import functools

import jax
import jax.numpy as jnp
from jax import lax
from jax.experimental import pallas as pl
from jax.experimental.pallas import tpu as pltpu
from jax.experimental.pallas import tpu_sc as plsc

K = 20
B, N, D = 4, 1024, 3
BN = B * N
RB = 512
NEG = -3.0e38
NW = 32
NPW = BN // NW


def _k1_body(x_ref, xt_ref, w_ref, wb_ref, wd_ref, b1_ref,
             idx_ref, p_ref, q_ref):
    b = pl.program_id(0)
    rb = pl.program_id(1)
    xr = x_ref[0]
    xt = xt_ref[0]
    d2 = jnp.zeros((RB, N), jnp.float32)
    for d in range(D):
        diff = xr[:, d:d + 1] - xt[d:d + 1, :]
        d2 = d2 + diff * diff
    rows = jax.lax.broadcasted_iota(jnp.int32, (RB, N), 0) + rb * RB
    cols = jax.lax.broadcasted_iota(jnp.int32, (RB, N), 1)
    d2 = jnp.where(rows == cols, d2 + 1e10, d2)
    neg = -d2
    r_cols = jnp.sqrt(jnp.sum(xt * xt, axis=0, keepdims=True))
    r_i = jnp.sqrt(jnp.sum(xr * xr, axis=1, keepdims=True))
    ri2 = r_i * r_i
    js, ds, rs = [], [], []
    for _ in range(K):
        m = jnp.max(neg, axis=1, keepdims=True)
        eq = neg == m
        j = jnp.min(jnp.where(eq, cols, jnp.int32(2 ** 30)), axis=1,
                    keepdims=True)
        sel = cols == j
        rj = jnp.max(jnp.where(sel, r_cols, NEG), axis=1, keepdims=True)
        neg = jnp.where(sel, NEG, neg)
        js.append(j)
        ds.append(-m)
        rs.append(rj)
    d2k = jnp.concatenate(ds, axis=1)
    rjm = jnp.concatenate(rs, axis=1)
    dist = jnp.sqrt(d2k)
    dot = 0.5 * (ri2 + rjm * rjm - d2k)
    cos = dot / (r_i * rjm + 1e-8)
    cos = jnp.clip(cos, -1.0 + 1e-7, 1.0 - 1e-7)
    theta = 2.0 * jnp.arctan2(jnp.sqrt((1.0 - cos) * (1.0 + cos)),
                              1.0 + cos)
    f = jnp.concatenate([r_i, rjm, theta, dist], axis=1)
    g = jnp.dot(f, w_ref[...], preferred_element_type=jnp.float32)
    h = g[:, :64]
    for k in range(1, K):
        h = jnp.maximum(h, g[:, 64 * k:64 * (k + 1)])
    idx_ref[0] = jnp.concatenate(js, axis=1) + b * N
    p_ref[0] = jnp.dot(h, wb_ref[...], preferred_element_type=jnp.float32)
    q_ref[0] = jnp.dot(h, wd_ref[...],
                       preferred_element_type=jnp.float32) + b1_ref[...]


def _k1(x, xt, w_rri, w1b, w1d, b1):
    return pl.pallas_call(
        _k1_body,
        grid=(B, N // RB),
        in_specs=[
            pl.BlockSpec((1, RB, D), lambda b, r: (b, r, 0)),
            pl.BlockSpec((1, D, N), lambda b, r: (b, 0, 0)),
            pl.BlockSpec((3 * K + 1, 64 * K), lambda b, r: (0, 0)),
            pl.BlockSpec((64, 128), lambda b, r: (0, 0)),
            pl.BlockSpec((64, 128), lambda b, r: (0, 0)),
            pl.BlockSpec((1, 128), lambda b, r: (0, 0)),
        ],
        out_specs=[
            pl.BlockSpec((1, RB, K), lambda b, r: (b, r, 0)),
            pl.BlockSpec((1, RB, 128), lambda b, r: (b, r, 0)),
            pl.BlockSpec((1, RB, 128), lambda b, r: (b, r, 0)),
        ],
        out_shape=[
            jax.ShapeDtypeStruct((B, N, K), jnp.int32),
            jax.ShapeDtypeStruct((B, N, 128), jnp.float32),
            jax.ShapeDtypeStruct((B, N, 128), jnp.float32),
        ],
    )(x, xt, w_rri, w1b, w1d, b1)


GN3 = 16
GE3 = GN3 * K
NG3 = NPW // GN3


def _k3_sc_body(p_hbm, idx_hbm, out_hbm, raw_v, rows_a, rows_b, out_v,
                sem_a, sem_b):
    wid = lax.axis_index("s") * 2 + lax.axis_index("c")
    base = wid * NPW
    pltpu.sync_copy(idx_hbm.at[pl.ds(base * K, NPW * K)], raw_v)
    bufs = (rows_a, rows_b)
    sems = (sem_a, sem_b)

    def fire(g):
        buf, sem = bufs[g % 2], sems[g % 2]
        return [
            pltpu.async_copy(
                p_hbm.at[raw_v.at[pl.ds(g * GE3 + c * 64, 64)]],
                buf.at[pl.ds(c * 64, 64)], sem)
            for c in range(GE3 // 64)
        ]

    cps = {0: fire(0)}
    for g in range(NG3):
        if g + 1 < NG3:
            cps[g + 1] = fire(g + 1)
        for cp in cps[g]:
            cp.wait()
        buf = bufs[g % 2]

        def body(m, _):
            r0 = m * K
            for c in range(8):
                sl = pl.ds(c * 16, 16)
                acc = buf[r0, sl]
                for e in range(1, K):
                    acc = jnp.maximum(acc, buf[r0 + e, sl])
                out_v[g * GN3 + m, sl] = acc
            return 0

        lax.fori_loop(0, GN3, body, 0)
    pltpu.sync_copy(out_v, out_hbm.at[pl.ds(base, NPW)])


def _k3_sc(p, idxt):
    mesh = plsc.VectorSubcoreMesh(core_axis_name="c", subcore_axis_name="s")
    f = functools.partial(
        pl.kernel,
        out_type=jax.ShapeDtypeStruct((BN, 128), jnp.float32),
        mesh=mesh,
        scratch_types=[
            pltpu.VMEM((NPW * K,), jnp.int32),
            pltpu.VMEM((GE3, 128), jnp.float32),
            pltpu.VMEM((GE3, 128), jnp.float32),
            pltpu.VMEM((NPW, 128), jnp.float32),
            pltpu.SemaphoreType.DMA,
            pltpu.SemaphoreType.DMA,
        ],
    )(_k3_sc_body)
    return f(p, idxt)


GN5 = 8
GE5 = GN5 * K
NG5 = NPW // GN5


def _k5_sc_body(ss_hbm, idx_hbm, num_hbm, den_hbm,
                raw_v, rows_a, rows_b, own_v, res_v, sem_a, sem_b):
    wid = lax.axis_index("s") * 2 + lax.axis_index("c")
    base = wid * NPW
    pltpu.sync_copy(idx_hbm.at[pl.ds(base * K, NPW * K)], raw_v)
    pltpu.sync_copy(ss_hbm.at[pl.ds(base, NPW)], own_v)
    bufs = (rows_a, rows_b)
    sems = (sem_a, sem_b)

    def fire(g):
        buf, sem = bufs[g % 2], sems[g % 2]
        return [
            pltpu.async_copy(
                ss_hbm.at[raw_v.at[pl.ds(g * GE5 + c * 80, 80)]],
                buf.at[pl.ds(c * 80, 80)], sem)
            for c in range(GE5 // 80)
        ]

    an = jnp.zeros((16,), jnp.float32)
    ad = jnp.zeros((16,), jnp.float32)
    cps = {0: fire(0)}
    for g in range(NG5):
        if g + 1 < NG5:
            cps[g + 1] = fire(g + 1)
        for cp in cps[g]:
            cp.wait()
        buf = bufs[g % 2]

        def body(m, carry):
            an_, ad_ = carry
            r0 = m * K
            o0 = own_v[g * GN5 + m, pl.ds(0, 16)]
            o1 = own_v[g * GN5 + m, pl.ds(16, 16)]
            for e in range(K):
                g0 = buf[r0 + e, pl.ds(0, 16)]
                g1 = buf[r0 + e, pl.ds(16, 16)]
                an_ = an_ + g0 * o0 + g1 * o1
                ad_ = ad_ + g0 * g0 + g1 * g1
            return an_, ad_

        an, ad = lax.fori_loop(0, GN5, body, (an, ad))
    res_v[0, :] = an
    res_v[1, :] = ad
    pltpu.sync_copy(res_v.at[0], num_hbm.at[wid])
    pltpu.sync_copy(res_v.at[1], den_hbm.at[wid])


def _k5_sc(ssoft, idxt):
    mesh = plsc.VectorSubcoreMesh(core_axis_name="c", subcore_axis_name="s")
    f = functools.partial(
        pl.kernel,
        out_type=[
            jax.ShapeDtypeStruct((NW, 16), jnp.float32),
            jax.ShapeDtypeStruct((NW, 16), jnp.float32),
        ],
        mesh=mesh,
        scratch_types=[
            pltpu.VMEM((NPW * K,), jnp.int32),
            pltpu.VMEM((GE5, 128), jnp.float32),
            pltpu.VMEM((GE5, 128), jnp.float32),
            pltpu.VMEM((NPW, 128), jnp.float32),
            pltpu.VMEM((2, 16), jnp.float32),
            pltpu.SemaphoreType.DMA,
            pltpu.SemaphoreType.DMA,
        ],
    )(_k5_sc_body)
    return f(ssoft, idxt)


def _ortho_stats(ssoft, csize):
    ss = lax.dot_general(ssoft, ssoft, (((0,), (0,)), ((), ())),
                         preferred_element_type=jnp.float32)
    frob = jnp.sqrt(jnp.sum(ss * ss, keepdims=True)) + 1e-15
    ceye = jnp.where(
        jax.lax.broadcasted_iota(jnp.int32, (csize, csize), 0)
        == jax.lax.broadcasted_iota(jnp.int32, (csize, csize), 1),
        1.0 / jnp.sqrt(jnp.float32(csize)), 0.0)
    diff = ss / frob - ceye
    return jnp.sqrt(jnp.sum(diff * diff, keepdims=True))


def _k4_body(mg_ref, q_ref, wp_ref, bp_ref, pool_ref, ssoft_ref, o_ref):
    xd = mg_ref[0] + q_ref[0]
    s = jnp.dot(xd, wp_ref[...], preferred_element_type=jnp.float32) + bp_ref[...]
    s = s - jnp.max(s, axis=1, keepdims=True)
    e = jnp.exp(s)
    ssoft = e / jnp.sum(e, axis=1, keepdims=True)
    pool_ref[0] = lax.dot_general(ssoft, xd, (((0,), (0,)), ((), ())),
                                  preferred_element_type=jnp.float32)
    ssoft_ref[0] = jnp.concatenate(
        [ssoft, jnp.zeros((N, 96), jnp.float32)], axis=1)
    o_ref[0] = _ortho_stats(ssoft, 32)


def _k4(mg, q, wp1, bp1):
    return pl.pallas_call(
        _k4_body,
        grid=(B,),
        in_specs=[
            pl.BlockSpec((1, N, 128), lambda b: (b, 0, 0)),
            pl.BlockSpec((1, N, 128), lambda b: (b, 0, 0)),
            pl.BlockSpec((128, 32), lambda b: (0, 0)),
            pl.BlockSpec((1, 32), lambda b: (0, 0)),
        ],
        out_specs=[
            pl.BlockSpec((1, 32, 128), lambda b: (b, 0, 0)),
            pl.BlockSpec((1, N, 128), lambda b: (b, 0, 0)),
            pl.BlockSpec((1, 1, 1), lambda b: (b, 0, 0)),
        ],
        out_shape=[
            jax.ShapeDtypeStruct((B, 32, 128), jnp.float32),
            jax.ShapeDtypeStruct((B, N, 128), jnp.float32),
            jax.ShapeDtypeStruct((B, 1, 1), jnp.float32),
        ],
    )(mg, q, wp1, bp1)


def _pool_stats(ssoft, m, csize):
    t = jnp.dot(m, ssoft, preferred_element_type=jnp.float32)
    num = jnp.sum(t * ssoft, keepdims=True)
    ssq_row = jnp.sum(ssoft * ssoft, axis=1, keepdims=True)
    deg = jnp.sum(m, axis=0, keepdims=True)
    den = jnp.dot(deg, ssq_row, preferred_element_type=jnp.float32) + 1e-15
    o = _ortho_stats(ssoft, csize)
    return num, den, o


def _k6_body(x_ref, wb_ref, wd_ref, b_ref, wp_ref, bp_ref,
             pool_ref, num_ref, den_ref, o_ref):
    x = x_ref[0]
    n = 32
    diff = x[:, None, :] - x[None, :, :]
    d2 = jnp.sum(diff * diff, axis=-1)
    rows = jax.lax.broadcasted_iota(jnp.int32, (n, n), 0)
    cols = jax.lax.broadcasted_iota(jnp.int32, (n, n), 1)
    d2 = jnp.where(rows == cols, d2 + 1e10, d2)
    neg = -d2
    macc = jnp.zeros((n, n), jnp.float32)
    for _ in range(K):
        m = jnp.max(neg, axis=1, keepdims=True)
        eq = neg == m
        j = jnp.min(jnp.where(eq, cols, jnp.int32(2 ** 30)), axis=1,
                    keepdims=True)
        sel = cols == j
        neg = jnp.where(sel, NEG, neg)
        macc = jnp.where(sel, 1.0, macc)
    p = jnp.dot(x, wb_ref[...], preferred_element_type=jnp.float32)
    q = jnp.dot(x, wd_ref[...], preferred_element_type=jnp.float32) + b_ref[...]
    mx = jnp.max(jnp.where(macc[:, :, None] > 0, p[None, :, :], NEG), axis=1)
    h2 = q + mx
    s = jnp.dot(h2, wp_ref[...], preferred_element_type=jnp.float32) + bp_ref[...]
    s = s - jnp.max(s, axis=1, keepdims=True)
    e = jnp.exp(s)
    ssoft = e / jnp.sum(e, axis=1, keepdims=True)
    pool_ref[0] = lax.dot_general(ssoft, h2, (((0,), (0,)), ((), ())),
                                  preferred_element_type=jnp.float32)
    num, den, o = _pool_stats(ssoft, macc, 8)
    num_ref[0] = num
    den_ref[0] = den
    o_ref[0] = o


def _k6(xd, w2b, w2d, b2, wp2, bp2):
    return pl.pallas_call(
        _k6_body,
        grid=(B,),
        in_specs=[
            pl.BlockSpec((1, 32, 128), lambda b: (b, 0, 0)),
            pl.BlockSpec((128, 256), lambda b: (0, 0)),
            pl.BlockSpec((128, 256), lambda b: (0, 0)),
            pl.BlockSpec((1, 256), lambda b: (0, 0)),
            pl.BlockSpec((256, 8), lambda b: (0, 0)),
            pl.BlockSpec((1, 8), lambda b: (0, 0)),
        ],
        out_specs=[
            pl.BlockSpec((1, 8, 256), lambda b: (b, 0, 0)),
            pl.BlockSpec((1, 1, 1), lambda b: (b, 0, 0)),
            pl.BlockSpec((1, 1, 1), lambda b: (b, 0, 0)),
            pl.BlockSpec((1, 1, 1), lambda b: (b, 0, 0)),
        ],
        out_shape=[
            jax.ShapeDtypeStruct((B, 8, 256), jnp.float32),
            jax.ShapeDtypeStruct((B, 1, 1), jnp.float32),
            jax.ShapeDtypeStruct((B, 1, 1), jnp.float32),
            jax.ShapeDtypeStruct((B, 1, 1), jnp.float32),
        ],
    )(xd, w2b, w2d, b2, wp2, bp2)


def _k7_body(x_ref, wb_ref, wd_ref, b_ref, wf1_ref, bf1_ref, wf2_ref,
             bf2_ref, wf3_ref, bf3_ref, out_ref):
    x = x_ref[...].reshape(B * 8, 256)
    p = jnp.dot(x, wb_ref[...], preferred_element_type=jnp.float32)
    q = jnp.dot(x, wd_ref[...], preferred_element_type=jnp.float32) + b_ref[...]
    pooled_rows = []
    rows3 = jax.lax.broadcasted_iota(jnp.int32, (8, 8, 1024), 0)
    cols3 = jax.lax.broadcasted_iota(jnp.int32, (8, 8, 1024), 1)
    offdiag = rows3 != cols3
    for b in range(B):
        pb = p[b * 8:(b + 1) * 8, :]
        mx = jnp.max(jnp.where(offdiag, pb[None, :, :], NEG),
                     axis=1)
        h3 = q[b * 8:(b + 1) * 8, :] + mx
        pooled_rows.append(jnp.sum(h3, axis=0, keepdims=True))
    pooled = jnp.concatenate(pooled_rows, axis=0)
    h4 = jnp.maximum(
        jnp.dot(pooled, wf1_ref[...], preferred_element_type=jnp.float32)
        + bf1_ref[...], 0.0)
    h5 = jnp.maximum(
        jnp.dot(h4, wf2_ref[...], preferred_element_type=jnp.float32)
        + bf2_ref[...], 0.0)
    h6 = jnp.dot(h5, wf3_ref[...], preferred_element_type=jnp.float32) \
        + bf3_ref[...]
    h6 = h6 - jnp.max(h6, axis=1, keepdims=True)
    e = jnp.exp(h6)
    out_ref[...] = e / jnp.sum(e, axis=1, keepdims=True)


def _k7(xd, w3b, w3d, b3, wf1, bf1, wf2, bf2, wf3, bf3):
    full = lambda *shape: pl.BlockSpec(shape, lambda: tuple(0 for _ in shape))
    return pl.pallas_call(
        _k7_body,
        grid=(),
        in_specs=[
            full(B, 8, 256),
            full(256, 1024), full(256, 1024), full(1, 1024),
            full(1024, 512), full(1, 512),
            full(512, 256), full(1, 256),
            full(256, 40), full(1, 40),
        ],
        out_specs=full(B, 40),
        out_shape=jax.ShapeDtypeStruct((B, 40), jnp.float32),
    )(xd, w3b, w3d, b3, wf1, bf1, wf2, bf2, wf3, bf3)


def kernel(x, W_rri, W1, b1, Wp1, bp1, W2, b2, Wp2, bp2, W3, b3, Wp3, bp3,
           Wf1, bf1, Wf2, bf2, Wf3, bf3):
    xt = jnp.transpose(x, (0, 2, 1))
    w1b, w1d = W1[64:], W1[:64] - W1[64:]
    eyek = jnp.eye(K, dtype=jnp.float32)
    w_big = jnp.concatenate(
        [jnp.tile(W_rri[0:1, :], (1, K)),
         jnp.kron(eyek, W_rri[1:2, :]),
         jnp.kron(eyek, W_rri[2:3, :]),
         jnp.kron(eyek, W_rri[3:4, :])], axis=0)
    idx, p1, q1 = _k1(x, xt, w_big, w1b, w1d, b1.reshape(1, -1))
    idxt = idx.reshape(BN * K)
    mg = _k3_sc(p1.reshape(BN, 128), idxt)
    pool1, ssoft1, o1 = _k4(mg.reshape(B, N, 128), q1, Wp1, bp1.reshape(1, -1))
    num1, den1 = _k5_sc(ssoft1.reshape(BN, 128), idxt)
    w2b, w2d = W2[128:], W2[:128] - W2[128:]
    pool2, num2, den2, o2 = _k6(pool1, w2b, w2d, b2.reshape(1, -1), Wp2,
                                bp2.reshape(1, -1))
    w3b, w3d = W3[256:], W3[:256] - W3[256:]
    out = _k7(pool2, w3b, w3d, b3.reshape(1, -1), Wf1, bf1.reshape(1, -1),
              Wf2, bf2.reshape(1, -1), Wf3, bf3.reshape(1, -1))
    num1_b = jnp.sum(num1.reshape(B, 8 * 16), axis=1)
    den1_b = jnp.sum(den1.reshape(B, 8 * 16), axis=1) + 1e-15
    mc = (jnp.mean(-(num1_b / den1_b))
          + jnp.mean(-(num2[:, 0, 0] / den2[:, 0, 0])) - 1.0)
    o = jnp.mean(o1[:, 0, 0]) + jnp.mean(o2[:, 0, 0]) + 0.0
    return out, mc, o
